# baseline (speedup 1.0000x reference)
import functools

import jax
import jax.numpy as jnp
import numpy as np
from jax import lax
from jax.experimental import pallas as pl
from jax.experimental.pallas import tpu as pltpu
from jax.experimental.pallas import tpu_sc as plsc

_RES = np.float32(0.1)
_MASK = np.int32(1048576 - 1)
_P0 = np.int32(73856093)
_P1 = np.int32(19349669)
_P2 = np.int32(83492791)
_L = 16
_NW = 32
_CHUNK = 128


def _floor_res(q):
    t = q / _RES
    i = t.astype(jnp.int32)
    f = i.astype(jnp.float32)
    return jnp.where(f > t, i - np.int32(1), i)


@functools.lru_cache(maxsize=None)
def _make_kernel(n, d, zero_row):
    pw = n // _NW
    n_chunks = pw // _CHUNK
    mesh = plsc.VectorSubcoreMesh(core_axis_name="c", subcore_axis_name="s")

    @functools.partial(
        pl.kernel,
        mesh=mesh,
        compiler_params=pltpu.CompilerParams(use_tc_tiling_on_sc=False),
        out_type=jax.ShapeDtypeStruct((n, d), jnp.float32),
        scratch_types=[
            pltpu.VMEM((_CHUNK,), jnp.float32),
            pltpu.VMEM((_CHUNK,), jnp.float32),
            pltpu.VMEM((_CHUNK,), jnp.float32),
            pltpu.VMEM((_CHUNK,), jnp.int32),
            pltpu.VMEM((_CHUNK,), jnp.int32),
            pltpu.VMEM((_CHUNK,), jnp.int32),
            pltpu.VMEM((_CHUNK, d), jnp.float32),
            pltpu.SemaphoreType.DMA,
        ],
    )
    def k(qx_h, qy_h, qz_h, buf_h, feat_h, out_h,
          qx_v, qy_v, qz_v, hash_v, vox_v, idx_v, rows_v, sem):
        wid = lax.axis_index("s") * np.int32(2) + lax.axis_index("c")
        base = wid * np.int32(pw)

        def chunk_body(step, off):
            off = pl.multiple_of(off, _CHUNK)
            pltpu.sync_copy(qx_h.at[pl.ds(off, _CHUNK)], qx_v)
            pltpu.sync_copy(qy_h.at[pl.ds(off, _CHUNK)], qy_v)
            pltpu.sync_copy(qz_h.at[pl.ds(off, _CHUNK)], qz_v)
            for j in range(_CHUNK // _L):
                sl = pl.ds(j * _L, _L)
                gx = _floor_res(qx_v[sl])
                gy = _floor_res(qy_v[sl])
                gz = _floor_res(qz_v[sl])
                hash_v[sl] = (gx * _P0 + gy * _P1 + gz * _P2) & _MASK
            pltpu.async_copy(buf_h.at[hash_v], vox_v, sem).wait()
            for j in range(_CHUNK // _L):
                sl = pl.ds(j * _L, _L)
                v = vox_v[sl]
                idx_v[sl] = jnp.where(v >= np.int32(0), v, np.int32(zero_row))
            pltpu.async_copy(feat_h.at[idx_v], rows_v, sem).wait()
            pltpu.sync_copy(rows_v, out_h.at[pl.ds(off, _CHUNK)])
            return off + np.int32(_CHUNK)

        lax.fori_loop(0, n_chunks, chunk_body, base)

    return k


def kernel(query_pts, features, buffer_voxel_index):
    n = query_pts.shape[0]
    nv, d = features.shape
    qt = query_pts.T
    qx, qy, qz = qt[0], qt[1], qt[2]
    buf = buffer_voxel_index.astype(jnp.int32)
    feat_ext = jnp.concatenate(
        [features.astype(jnp.float32), jnp.zeros((1, d), jnp.float32)], axis=0)
    return _make_kernel(n, d, nv)(qx, qy, qz, buf, feat_ext)
